# Initial kernel scaffold; baseline (speedup 1.0000x reference)
import jax
import jax.numpy as jnp
from jax.experimental import pallas as pl


def kernel(image_sizes, anchors, class_logits, bbox_deltas):
    raise NotImplementedError("write your pallas kernel here")


# speedup vs baseline: 4.8773x; 4.8773x over previous
import functools

import jax
import jax.numpy as jnp
from jax.experimental import pallas as pl
from jax.experimental.pallas import tpu as pltpu

_B = 8
_N = 20000
_NPAD = 20480
_PRE = 6000
_PREPAD = 6144
_POST = 1000
_THRESH = 0.7
_MINSZ = 16.0
_CHUNK = 128


def _decode_kernel(hm1_ref, wm1_ref, ax1_ref, ay1_ref, ax2_ref, ay2_ref,
                   tx_ref, ty_ref, tw_ref, th_ref,
                   px1_ref, py1_ref, px2_ref, py2_ref, valid_ref):
    a_w = ax2_ref[:] - ax1_ref[:]
    a_h = ay2_ref[:] - ay1_ref[:]
    a_x = ax1_ref[:] + 0.5 * a_w
    a_y = ay1_ref[:] + 0.5 * a_h
    p_x = a_x + tx_ref[:] * a_w
    p_y = a_y + ty_ref[:] * a_h
    p_w = a_w * jnp.exp(jnp.clip(tw_ref[:], -10.0, 10.0))
    p_h = a_h * jnp.exp(jnp.clip(th_ref[:], -10.0, 10.0))
    x1 = p_x - 0.5 * p_w
    y1 = p_y - 0.5 * p_h
    x2 = p_x + 0.5 * p_w
    y2 = p_y + 0.5 * p_h
    wm1 = wm1_ref[:]
    hm1 = hm1_ref[:]
    x1 = jnp.clip(x1, 0.0, wm1)
    x2 = jnp.clip(x2, 0.0, wm1)
    y1 = jnp.clip(y1, 0.0, hm1)
    y2 = jnp.clip(y2, 0.0, hm1)
    ws = x2 - x1
    hs = y2 - y1
    valid = (ws >= _MINSZ) & (hs >= _MINSZ)
    px1_ref[:] = x1
    py1_ref[:] = y1
    px2_ref[:] = x2
    py2_ref[:] = y2
    valid_ref[:] = valid.astype(jnp.float32)


def _nms_kernel(x1_ref, y1_ref, x2_ref, y2_ref, valid_ref, keep_ref, sup_ref):
    keep_ref[:] = jnp.zeros((_B, _PREPAD), jnp.float32)
    sup_ref[:] = jnp.zeros((_B, _PREPAD), jnp.float32)
    x1 = x1_ref[:]
    y1 = y1_ref[:]
    x2 = x2_ref[:]
    y2 = y2_ref[:]
    areas = (x2 - x1) * (y2 - y1)
    lane = jax.lax.broadcasted_iota(jnp.int32, (_B, _PREPAD), 1)

    def step(c, counts):
        onehot = lane == c

        def col(v):
            return jnp.sum(jnp.where(onehot, v, 0.0), axis=1, keepdims=True)

        sup = sup_ref[:]
        alive = col(valid_ref[:]) * (1.0 - col(sup))
        bx1 = col(x1)
        by1 = col(y1)
        bx2 = col(x2)
        by2 = col(y2)
        xx1 = jnp.maximum(bx1, x1)
        yy1 = jnp.maximum(by1, y1)
        xx2 = jnp.minimum(bx2, x2)
        yy2 = jnp.minimum(by2, y2)
        inter = jnp.maximum(xx2 - xx1, 0.0) * jnp.maximum(yy2 - yy1, 0.0)
        b_area = (bx2 - bx1) * (by2 - by1)
        iou = inter / (areas + b_area - inter + 1e-9)
        suppress = (iou > _THRESH) & (alive > 0.0)
        sup_ref[:] = jnp.where(suppress, 1.0, sup)
        keep_ref[:] = jnp.where(onehot & (alive > 0.0), 1.0, keep_ref[:])
        return counts + alive

    def outer_cond(carry):
        k, counts = carry
        return (k < _PREPAD // _CHUNK) & (jnp.min(counts) < float(_POST))

    def outer_body(carry):
        k, counts = carry
        counts = jax.lax.fori_loop(
            0, _CHUNK, lambda i, cc: step(k * _CHUNK + i, cc), counts)
        return k + 1, counts

    jax.lax.while_loop(
        outer_cond, outer_body, (0, jnp.zeros((_B, 1), jnp.float32)))


def kernel(image_sizes, anchors, class_logits, bbox_deltas):
    f32 = jnp.float32
    pad = ((0, 0), (0, _NPAD - _N))
    ax1 = jnp.pad(anchors[:, :, 0], pad)
    ay1 = jnp.pad(anchors[:, :, 1], pad)
    ax2 = jnp.pad(anchors[:, :, 2], pad)
    ay2 = jnp.pad(anchors[:, :, 3], pad)
    tx = jnp.pad(bbox_deltas[:, :, 0], pad)
    ty = jnp.pad(bbox_deltas[:, :, 1], pad)
    tw = jnp.pad(bbox_deltas[:, :, 2], pad)
    th = jnp.pad(bbox_deltas[:, :, 3], pad)
    logits = jnp.pad(class_logits[:, :, 0], pad)
    hm1 = image_sizes[:, 0:1].astype(f32) - 1.0
    wm1 = image_sizes[:, 1:2].astype(f32) - 1.0

    plane = jax.ShapeDtypeStruct((_B, _NPAD), f32)
    px1, py1, px2, py2, valid = pl.pallas_call(
        _decode_kernel,
        out_shape=[plane, plane, plane, plane, plane],
    )(hm1, wm1, ax1, ay1, ax2, ay2, tx, ty, tw, th)

    scores = jnp.where(valid > 0.0, jax.nn.sigmoid(logits), -jnp.inf)
    scores = jnp.where(jnp.arange(_NPAD)[None, :] < _N, scores, -jnp.inf)
    top_scores, top_idx = jax.lax.top_k(scores, _PRE)

    cpad = ((0, 0), (0, _PREPAD - _PRE))
    sx1 = jnp.pad(jnp.take_along_axis(px1, top_idx, axis=1), cpad)
    sy1 = jnp.pad(jnp.take_along_axis(py1, top_idx, axis=1), cpad)
    sx2 = jnp.pad(jnp.take_along_axis(px2, top_idx, axis=1), cpad)
    sy2 = jnp.pad(jnp.take_along_axis(py2, top_idx, axis=1), cpad)
    svalid = jnp.pad(
        jnp.isfinite(top_scores).astype(f32), cpad)

    splane = jax.ShapeDtypeStruct((_B, _PREPAD), f32)
    keep, _ = pl.pallas_call(
        _nms_kernel,
        out_shape=[splane, splane],
    )(sx1, sy1, sx2, sy2, svalid)

    keepb = keep > 0.0
    csum = jnp.cumsum(keepb.astype(jnp.int32), axis=1)
    pos = jnp.where(keepb, csum - 1, _POST)
    pos = jnp.minimum(pos, _POST)
    rows = jnp.arange(_B)[:, None]
    boxes = jnp.stack([sx1, sy1, sx2, sy2], axis=-1)
    out = jnp.zeros((_B, _POST + 1, 4), f32).at[rows, pos].set(
        boxes, mode="drop", unique_indices=False)
    keep_boxes = out[:, :_POST]
    counts = csum[:, -1]
    keep_mask = jnp.arange(_POST)[None, :] < counts[:, None]
    return keep_boxes, keep_mask


# speedup vs baseline: 7.6206x; 1.5624x over previous
import functools

import jax
import jax.numpy as jnp
from jax.experimental import pallas as pl
from jax.experimental.pallas import tpu as pltpu

_B = 8
_N = 20000
_NPAD = 20480
_PRE = 6000
_PREPAD = 6144
_POST = 1000
_POSTPAD = 1024
_THRESH = 0.7
_MINSZ = 16.0
_CHUNK = 128


def _decode_kernel(hm1_ref, wm1_ref, a_ref, d_ref,
                   px1_ref, py1_ref, px2_ref, py2_ref, valid_ref):
    ax1 = a_ref[:, 0, :]
    ay1 = a_ref[:, 1, :]
    ax2 = a_ref[:, 2, :]
    ay2 = a_ref[:, 3, :]
    a_w = ax2 - ax1
    a_h = ay2 - ay1
    a_x = ax1 + 0.5 * a_w
    a_y = ay1 + 0.5 * a_h
    p_x = a_x + d_ref[:, 0, :] * a_w
    p_y = a_y + d_ref[:, 1, :] * a_h
    p_w = a_w * jnp.exp(jnp.clip(d_ref[:, 2, :], -10.0, 10.0))
    p_h = a_h * jnp.exp(jnp.clip(d_ref[:, 3, :], -10.0, 10.0))
    x1 = p_x - 0.5 * p_w
    y1 = p_y - 0.5 * p_h
    x2 = p_x + 0.5 * p_w
    y2 = p_y + 0.5 * p_h
    wm1 = wm1_ref[:]
    hm1 = hm1_ref[:]
    x1 = jnp.clip(x1, 0.0, wm1)
    x2 = jnp.clip(x2, 0.0, wm1)
    y1 = jnp.clip(y1, 0.0, hm1)
    y2 = jnp.clip(y2, 0.0, hm1)
    ws = x2 - x1
    hs = y2 - y1
    valid = (ws >= _MINSZ) & (hs >= _MINSZ)
    px1_ref[:] = x1
    py1_ref[:] = y1
    px2_ref[:] = x2
    py2_ref[:] = y2
    valid_ref[:] = valid.astype(jnp.float32)


def _nms_kernel(x1_ref, y1_ref, x2_ref, y2_ref, valid_ref,
                ox1_ref, oy1_ref, ox2_ref, oy2_ref, cnt_ref, sup_ref):
    ox1_ref[:] = jnp.zeros((_B, _POSTPAD), jnp.float32)
    oy1_ref[:] = jnp.zeros((_B, _POSTPAD), jnp.float32)
    ox2_ref[:] = jnp.zeros((_B, _POSTPAD), jnp.float32)
    oy2_ref[:] = jnp.zeros((_B, _POSTPAD), jnp.float32)
    sup_ref[:] = 1.0 - valid_ref[:]
    x1 = x1_ref[:]
    y1 = y1_ref[:]
    x2 = x2_ref[:]
    y2 = y2_ref[:]
    areas = (x2 - x1) * (y2 - y1)
    lane_out = jax.lax.broadcasted_iota(jnp.int32, (_B, _POSTPAD), 1)
    lane_ch = jax.lax.broadcasted_iota(jnp.int32, (_B, _CHUNK), 1)

    def step(kb, cx1, cy1, cx2, cy2, i, counts):
        oh = lane_ch == i

        def col(v):
            return jnp.sum(jnp.where(oh, v, 0.0), axis=1, keepdims=True)

        sup_ch = sup_ref[:, pl.ds(kb, _CHUNK)]
        alive = 1.0 - col(sup_ch)
        bx1 = col(cx1)
        by1 = col(cy1)
        bx2 = col(cx2)
        by2 = col(cy2)
        xx1 = jnp.maximum(bx1, x1)
        yy1 = jnp.maximum(by1, y1)
        xx2 = jnp.minimum(bx2, x2)
        yy2 = jnp.minimum(by2, y2)
        inter = jnp.maximum(xx2 - xx1, 0.0) * jnp.maximum(yy2 - yy1, 0.0)
        b_area = (bx2 - bx1) * (by2 - by1)
        iou = inter / (areas + b_area - inter + 1e-9)
        sup_ref[:] = jnp.maximum(
            sup_ref[:], jnp.where(iou > _THRESH, alive, 0.0))
        slot = lane_out == counts.astype(jnp.int32)
        w = slot & (alive > 0.0)
        ox1_ref[:] = jnp.where(w, bx1, ox1_ref[:])
        oy1_ref[:] = jnp.where(w, by1, oy1_ref[:])
        ox2_ref[:] = jnp.where(w, bx2, ox2_ref[:])
        oy2_ref[:] = jnp.where(w, by2, oy2_ref[:])
        return counts + alive

    def outer_cond(carry):
        k, counts = carry
        return (k < _PREPAD // _CHUNK) & (jnp.min(counts) < float(_POST))

    def outer_body(carry):
        k, counts = carry
        kb = pl.multiple_of(k * _CHUNK, _CHUNK)
        cx1 = x1_ref[:, pl.ds(kb, _CHUNK)]
        cy1 = y1_ref[:, pl.ds(kb, _CHUNK)]
        cx2 = x2_ref[:, pl.ds(kb, _CHUNK)]
        cy2 = y2_ref[:, pl.ds(kb, _CHUNK)]
        counts = jax.lax.fori_loop(
            0, _CHUNK,
            lambda i, cc: step(kb, cx1, cy1, cx2, cy2, i, cc), counts)
        return k + 1, counts

    _, counts = jax.lax.while_loop(
        outer_cond, outer_body, (0, jnp.zeros((_B, 1), jnp.float32)))
    cnt_ref[:] = counts


def kernel(image_sizes, anchors, class_logits, bbox_deltas):
    f32 = jnp.float32
    pad3 = ((0, 0), (0, 0), (0, _NPAD - _N))
    at = jnp.pad(anchors.transpose(0, 2, 1), pad3)
    dt = jnp.pad(bbox_deltas.transpose(0, 2, 1), pad3)
    logits = jnp.pad(class_logits[:, :, 0], ((0, 0), (0, _NPAD - _N)))
    hm1 = image_sizes[:, 0:1].astype(f32) - 1.0
    wm1 = image_sizes[:, 1:2].astype(f32) - 1.0

    plane = jax.ShapeDtypeStruct((_B, _NPAD), f32)
    px1, py1, px2, py2, valid = pl.pallas_call(
        _decode_kernel,
        out_shape=[plane, plane, plane, plane, plane],
    )(hm1, wm1, at, dt)

    scores = jnp.where(valid > 0.0, jax.nn.sigmoid(logits), -jnp.inf)
    scores = jnp.where(jnp.arange(_NPAD)[None, :] < _N, scores, -jnp.inf)
    top_scores, top_idx = jax.lax.top_k(scores, _PRE)

    cpad = ((0, 0), (0, _PREPAD - _PRE))
    sx1 = jnp.pad(jnp.take_along_axis(px1, top_idx, axis=1), cpad)
    sy1 = jnp.pad(jnp.take_along_axis(py1, top_idx, axis=1), cpad)
    sx2 = jnp.pad(jnp.take_along_axis(px2, top_idx, axis=1), cpad)
    sy2 = jnp.pad(jnp.take_along_axis(py2, top_idx, axis=1), cpad)
    svalid = jnp.pad(jnp.isfinite(top_scores).astype(f32), cpad)

    oplane = jax.ShapeDtypeStruct((_B, _POSTPAD), f32)
    ox1, oy1, ox2, oy2, cnt, _ = pl.pallas_call(
        _nms_kernel,
        out_shape=[oplane, oplane, oplane, oplane,
                   jax.ShapeDtypeStruct((_B, 1), f32),
                   jax.ShapeDtypeStruct((_B, _PREPAD), f32)],
    )(sx1, sy1, sx2, sy2, svalid)

    keep_boxes = jnp.stack(
        [ox1[:, :_POST], oy1[:, :_POST], ox2[:, :_POST], oy2[:, :_POST]],
        axis=-1)
    counts = cnt[:, 0].astype(jnp.int32)
    keep_mask = jnp.arange(_POST)[None, :] < counts[:, None]
    return keep_boxes, keep_mask
